# baseline (speedup 1.0000x reference)
import functools

import jax
import jax.numpy as jnp
from jax import lax
from jax.experimental import pallas as pl
from jax.experimental.pallas import tpu as pltpu
from jax.experimental.pallas import tpu_sc as plsc

N = 100000
D = 128
K = 5
B = 16384
G = 3 * B
NW = 32
PER_W = G // NW
CH = 128
NCH = PER_W // CH


def _sc_gather(state, last_t, idx3):
    mesh = plsc.VectorSubcoreMesh(core_axis_name="c", subcore_axis_name="s")

    @functools.partial(
        pl.kernel,
        out_type=(
            jax.ShapeDtypeStruct((G, D), jnp.float32),
            jax.ShapeDtypeStruct((G,), jnp.float32),
        ),
        mesh=mesh,
        scratch_types=[
            pltpu.VMEM((NCH, CH), jnp.int32),
            pltpu.VMEM((CH, D), jnp.float32),
            pltpu.VMEM((CH,), jnp.float32),
            pltpu.SemaphoreType.DMA,
            pltpu.SemaphoreType.DMA,
        ],
    )
    def k(state_hbm, lastt_hbm, idx_hbm, rows_out, lt_out, idx_v, rows_v, lt_v,
          sem_r, sem_l):
        wid = lax.axis_index("s") * 2 + lax.axis_index("c")
        pltpu.sync_copy(idx_hbm.at[wid], idx_v)
        for j in range(NCH):
            base = wid * PER_W + j * CH
            cr = pltpu.async_copy(state_hbm.at[idx_v.at[j]], rows_v, sem_r)
            cl = pltpu.async_copy(lastt_hbm.at[idx_v.at[j]], lt_v, sem_l)
            cr.wait()
            pltpu.sync_copy(rows_v, rows_out.at[pl.ds(base, CH)])
            cl.wait()
            pltpu.sync_copy(lt_v, lt_out.at[pl.ds(base, CH)])

    return k(state, last_t, idx3)


_RB = 1024
_NBLK = G // _RB


def _tc_body(ld_ref, rows_ref, lt_ref, t_ref, w1_ref, b1_ref, w2_ref, b2_ref,
             out_ref):
    x = rows_ref[...]
    ltv = lt_ref[0, 0, :]
    tv = t_ref[0, 0, :]
    dt = jnp.maximum(tv - ltv, 0.0)
    ld = ld_ref[0, 0]
    decay = jnp.log1p(jnp.exp(jnp.full(dt.shape, ld, jnp.float32)))
    gate = jnp.exp(-decay * dt)
    xg = x * gate[:, None]
    h = jnp.dot(xg, w1_ref[...], preferred_element_type=jnp.float32,
                precision=lax.Precision.HIGHEST) + b1_ref[0, :][None, :]
    h = jnp.maximum(h, 0.0)
    logits = jnp.dot(h, w2_ref[...], preferred_element_type=jnp.float32,
                     precision=lax.Precision.HIGHEST) + b2_ref[0, :][None, :]
    m = jnp.max(logits, axis=-1, keepdims=True)
    e = jnp.exp(logits - m)
    out_ref[...] = e / jnp.sum(e, axis=-1, keepdims=True)


def _tc_mlp(rows, lt_g, t_all, log_decay, W1, b1, W2, b2, interpret=False):
    lt3 = lt_g.reshape(_NBLK, 1, _RB)
    t3 = t_all.reshape(_NBLK, 1, _RB)
    ld = jnp.reshape(log_decay, (1, 1))
    return pl.pallas_call(
        _tc_body,
        grid=(_NBLK,),
        in_specs=[
            pl.BlockSpec(memory_space=pltpu.SMEM),
            pl.BlockSpec((_RB, D), lambda i: (i, 0)),
            pl.BlockSpec((1, 1, _RB), lambda i: (i, 0, 0)),
            pl.BlockSpec((1, 1, _RB), lambda i: (i, 0, 0)),
            pl.BlockSpec((D, D), lambda i: (0, 0)),
            pl.BlockSpec((1, D), lambda i: (0, 0)),
            pl.BlockSpec((D, K), lambda i: (0, 0)),
            pl.BlockSpec((1, K), lambda i: (0, 0)),
        ],
        out_specs=pl.BlockSpec((_RB, K), lambda i: (i, 0)),
        out_shape=jax.ShapeDtypeStruct((G, K), jnp.float32),
        interpret=interpret,
    )(ld, rows, lt3, t3, W1, b1.reshape(1, D), W2, b2.reshape(1, K))


def kernel(source_nodes, destination_nodes, negative_nodes, edge_times,
           edge_idxs, state, last_t, log_decay, W1, b1, W2, b2):
    idx3 = jnp.concatenate(
        [source_nodes, destination_nodes, negative_nodes]).reshape(NW, NCH, CH)
    rows, lt_g = _sc_gather(state, last_t, idx3)
    t_all = jnp.concatenate([edge_times, edge_times, edge_times])
    out = _tc_mlp(rows, lt_g, t_all, log_decay, W1, b1, W2, b2)
    return (out[:B], out[B:2 * B], out[2 * B:])


# speedup vs baseline: 2.2290x; 2.2290x over previous
import functools

import jax
import jax.numpy as jnp
from jax import lax
from jax.experimental import pallas as pl
from jax.experimental.pallas import tpu as pltpu
from jax.experimental.pallas import tpu_sc as plsc

N = 100000
D = 128
K = 5
B = 16384
G = 3 * B
NW = 32
PER_W = G // NW
CH = 128
NCH = PER_W // CH
NB = 3


def _sc_gather(state, last_t, idx3):
    mesh = plsc.VectorSubcoreMesh(core_axis_name="c", subcore_axis_name="s")

    @functools.partial(
        pl.kernel,
        out_type=(
            jax.ShapeDtypeStruct((G, D), jnp.float32),
            jax.ShapeDtypeStruct((NW, NCH, CH), jnp.float32),
        ),
        mesh=mesh,
        scratch_types=[
            pltpu.VMEM((NCH, CH), jnp.int32),
            pltpu.VMEM((NB, CH, D), jnp.float32),
            pltpu.VMEM((NCH, CH), jnp.float32),
            pltpu.SemaphoreType.DMA,
            pltpu.SemaphoreType.DMA,
            pltpu.SemaphoreType.DMA,
            pltpu.SemaphoreType.DMA,
        ],
    )
    def k(state_hbm, lastt_hbm, idx_hbm, rows_out, lt_out, idx_v, rows_v,
          lt_v, sem_lt, s0, s1, s2):
        sems = (s0, s1, s2)
        wid = lax.axis_index("s") * 2 + lax.axis_index("c")
        pltpu.sync_copy(idx_hbm.at[wid], idx_v)
        lt_cps = [
            pltpu.async_copy(lastt_hbm.at[idx_v.at[j]], lt_v.at[j], sem_lt)
            for j in range(NCH)
        ]
        gcp = [None] * NCH
        for j in range(NB - 1):
            gcp[j] = pltpu.async_copy(
                state_hbm.at[idx_v.at[j]], rows_v.at[j % NB], sems[j % NB])
        for j in range(NCH):
            nxt = j + NB - 1
            if nxt < NCH:
                gcp[nxt] = pltpu.async_copy(
                    state_hbm.at[idx_v.at[nxt]], rows_v.at[nxt % NB],
                    sems[nxt % NB])
            gcp[j].wait()
            pltpu.sync_copy(rows_v.at[j % NB],
                            rows_out.at[pl.ds(wid * PER_W + j * CH, CH)])
        for cp in lt_cps:
            cp.wait()
        pltpu.sync_copy(lt_v, lt_out.at[wid])

    return k(state, last_t, idx3)


_RB = 1024
_NBLK = G // _RB


def _tc_body(ld_ref, rows_ref, lt_ref, t_ref, w1t_ref, b1_ref, w2t_ref,
             b2_ref, out_ref):
    ltv = lt_ref[0, 0, :]
    tv = t_ref[0, 0, :]
    dt = jnp.maximum(tv - ltv, 0.0)
    ld = ld_ref[0, 0]
    decay = jnp.log1p(jnp.exp(jnp.full((128,), ld, jnp.float32)))[0]
    gate = jnp.exp(-decay * dt)
    x = rows_ref[...].astype(jnp.bfloat16)
    yt = lax.dot_general(w1t_ref[...], x, (((1,), (1,)), ((), ())),
                         preferred_element_type=jnp.float32)
    ht = jnp.maximum(yt * gate[None, :] + b1_ref[...], 0.0)
    logits_t = jnp.dot(w2t_ref[...], ht,
                       preferred_element_type=jnp.float32) + b2_ref[...]
    m = jnp.max(logits_t, axis=0, keepdims=True)
    e = jnp.exp(logits_t - m)
    out_ref[...] = e / jnp.sum(e, axis=0, keepdims=True)


def _tc_mlp(rows, lt_g, t_all, log_decay, W1, b1, W2, b2, interpret=False):
    lt3 = lt_g.reshape(_NBLK, 1, _RB)
    t3 = t_all.reshape(_NBLK, 1, _RB)
    ld = jnp.reshape(log_decay, (1, 1))
    w1t = W1.T.astype(jnp.bfloat16)
    w2t = W2.T
    b1c = b1.reshape(D, 1)
    b2c = b2.reshape(K, 1)
    return pl.pallas_call(
        _tc_body,
        grid=(_NBLK,),
        in_specs=[
            pl.BlockSpec(memory_space=pltpu.SMEM),
            pl.BlockSpec((_RB, D), lambda i: (i, 0)),
            pl.BlockSpec((1, 1, _RB), lambda i: (i, 0, 0)),
            pl.BlockSpec((1, 1, _RB), lambda i: (i, 0, 0)),
            pl.BlockSpec((D, D), lambda i: (0, 0)),
            pl.BlockSpec((D, 1), lambda i: (0, 0)),
            pl.BlockSpec((K, D), lambda i: (0, 0)),
            pl.BlockSpec((K, 1), lambda i: (0, 0)),
        ],
        out_specs=pl.BlockSpec((K, _RB), lambda i: (0, i)),
        out_shape=jax.ShapeDtypeStruct((K, G), jnp.float32),
        interpret=interpret,
    )(ld, rows, lt3, t3, w1t, b1c, w2t, b2c)


def kernel(source_nodes, destination_nodes, negative_nodes, edge_times,
           edge_idxs, state, last_t, log_decay, W1, b1, W2, b2):
    idx3 = jnp.concatenate(
        [source_nodes, destination_nodes, negative_nodes]).reshape(NW, NCH, CH)
    rows, lt_g = _sc_gather(state, last_t, idx3)
    t_all = jnp.concatenate([edge_times, edge_times, edge_times])
    out_t = _tc_mlp(rows, lt_g.reshape(G), t_all, log_decay, W1, b1, W2, b2)
    out = out_t.T
    return (out[:B], out[B:2 * B], out[2 * B:])
